# Initial kernel scaffold; baseline (speedup 1.0000x reference)
import jax
import jax.numpy as jnp
from jax.experimental import pallas as pl


def kernel(input_ids, attention_mask, position_ids, weight):
    raise NotImplementedError("write your pallas kernel here")


# speedup vs baseline: 1.3934x; 1.3934x over previous
import functools

import jax
import jax.numpy as jnp
from jax import lax
from jax.experimental import pallas as pl
from jax.experimental.pallas import tpu as pltpu
from jax.experimental.pallas import tpu_sc as plsc

D_MODEL = 1024
NUM_CORES = 2
NUM_SUBCORES = 16
NUM_WORKERS = NUM_CORES * NUM_SUBCORES
CHUNK = 32
NBUF = 2


def _make_gather(n_rows: int):
    per_w = n_rows // NUM_WORKERS
    n_chunks = per_w // CHUNK
    mesh = plsc.VectorSubcoreMesh(core_axis_name="c", subcore_axis_name="s")

    @functools.partial(
        pl.kernel,
        mesh=mesh,
        out_type=jax.ShapeDtypeStruct((n_rows, D_MODEL), jnp.float32),
        scratch_types=[
            pltpu.VMEM((n_chunks, CHUNK), jnp.int32),
            pltpu.VMEM((NBUF, CHUNK, D_MODEL), jnp.float32),
            pltpu.SemaphoreType.DMA,
        ],
    )
    def gather_kernel(ids_hbm, table_hbm, out_hbm, idx_v, rows_v, gsem):
        wid = lax.axis_index("s") * NUM_CORES + lax.axis_index("c")
        base = wid * per_w
        pltpu.sync_copy(ids_hbm.at[wid], idx_v)
        for i in range(n_chunks):
            buf = rows_v.at[i % NBUF]
            pltpu.async_copy(table_hbm.at[idx_v.at[i]], buf, gsem).wait()
            pltpu.sync_copy(buf, out_hbm.at[pl.ds(base + i * CHUNK, CHUNK)])

    return gather_kernel


def kernel(input_ids, attention_mask, position_ids, weight):
    b, s = input_ids.shape
    n = b * s
    ids = input_ids.reshape(NUM_WORKERS, n // NUM_WORKERS // CHUNK, CHUNK)
    ids = ids.astype(jnp.int32)
    out = _make_gather(n)(ids, weight)
    return (out.reshape(b, s, D_MODEL), attention_mask, position_ids)


# speedup vs baseline: 1.6045x; 1.1515x over previous
import functools

import jax
import jax.numpy as jnp
from jax import lax
from jax.experimental import pallas as pl
from jax.experimental.pallas import tpu as pltpu
from jax.experimental.pallas import tpu_sc as plsc

D_MODEL = 1024
NUM_CORES = 2
NUM_SUBCORES = 16
NUM_WORKERS = NUM_CORES * NUM_SUBCORES
CHUNK = 32
NBUF = 3


def _make_gather(n_rows: int):
    per_w = n_rows // NUM_WORKERS
    n_chunks = per_w // CHUNK
    mesh = plsc.VectorSubcoreMesh(core_axis_name="c", subcore_axis_name="s")

    @functools.partial(
        pl.kernel,
        mesh=mesh,
        out_type=jax.ShapeDtypeStruct((n_rows, D_MODEL), jnp.float32),
        scratch_types=[
            pltpu.VMEM((n_chunks, CHUNK), jnp.int32),
            pltpu.VMEM((NBUF, CHUNK, D_MODEL), jnp.float32),
            pltpu.SemaphoreType.DMA((NBUF,)),
            pltpu.SemaphoreType.DMA((NBUF,)),
        ],
    )
    def gather_kernel(ids_hbm, table_hbm, out_hbm, idx_v, rows_v, gsem, osem):
        wid = lax.axis_index("s") * NUM_CORES + lax.axis_index("c")
        base = wid * per_w
        pltpu.sync_copy(ids_hbm.at[wid], idx_v)
        g_copies = [None] * n_chunks
        o_copies = [None] * n_chunks
        for i in range(n_chunks + 1):
            if i < n_chunks:
                b = i % NBUF
                if i >= NBUF:
                    o_copies[i - NBUF].wait()
                g_copies[i] = pltpu.async_copy(
                    table_hbm.at[idx_v.at[i]], rows_v.at[b], gsem.at[b])
            if i >= 1:
                j = i - 1
                g_copies[j].wait()
                o_copies[j] = pltpu.async_copy(
                    rows_v.at[j % NBUF],
                    out_hbm.at[pl.ds(base + j * CHUNK, CHUNK)],
                    osem.at[j % NBUF])
        for j in range(n_chunks - NBUF, n_chunks):
            o_copies[j].wait()

    return gather_kernel


def kernel(input_ids, attention_mask, position_ids, weight):
    b, s = input_ids.shape
    n = b * s
    ids = input_ids.reshape(NUM_WORKERS, n // NUM_WORKERS // CHUNK, CHUNK)
    ids = ids.astype(jnp.int32)
    out = _make_gather(n)(ids, weight)
    return (out.reshape(b, s, D_MODEL), attention_mask, position_ids)


# speedup vs baseline: 1.6260x; 1.0134x over previous
import functools

import jax
import jax.numpy as jnp
from jax import lax
from jax.experimental import pallas as pl
from jax.experimental.pallas import tpu as pltpu
from jax.experimental.pallas import tpu_sc as plsc

D_MODEL = 1024
NUM_CORES = 2
NUM_SUBCORES = 16
NUM_WORKERS = NUM_CORES * NUM_SUBCORES
CHUNK = 32
NBUF = 3


def _make_gather(n_rows: int):
    per_w = n_rows // NUM_WORKERS
    n_chunks = per_w // CHUNK
    mesh = plsc.VectorSubcoreMesh(core_axis_name="c", subcore_axis_name="s")

    @functools.partial(
        pl.kernel,
        mesh=mesh,
        out_type=jax.ShapeDtypeStruct((n_rows, D_MODEL), jnp.float32),
        scratch_types=[
            pltpu.VMEM((n_chunks, CHUNK), jnp.int32),
            pltpu.VMEM((NBUF, CHUNK, D_MODEL), jnp.float32),
            pltpu.SemaphoreType.DMA((NBUF,)),
            pltpu.SemaphoreType.DMA((NBUF,)),
        ],
    )
    def gather_kernel(ids_hbm, table_hbm, out_hbm, idx_v, rows_v, gsem, osem):
        wid = lax.axis_index("s") * NUM_CORES + lax.axis_index("c")
        base = wid * per_w
        pltpu.sync_copy(ids_hbm.at[wid], idx_v)
        g_copies = [None] * n_chunks
        o_copies = [None] * n_chunks
        lead = NBUF - 1
        for i in range(n_chunks + lead):
            if i < n_chunks:
                b = i % NBUF
                if i >= NBUF:
                    o_copies[i - NBUF].wait()
                g_copies[i] = pltpu.async_copy(
                    table_hbm.at[idx_v.at[i]], rows_v.at[b], gsem.at[b])
            if i >= lead:
                j = i - lead
                g_copies[j].wait()
                o_copies[j] = pltpu.async_copy(
                    rows_v.at[j % NBUF],
                    out_hbm.at[pl.ds(base + j * CHUNK, CHUNK)],
                    osem.at[j % NBUF])
        for j in range(n_chunks - NBUF, n_chunks):
            o_copies[j].wait()

    return gather_kernel


def kernel(input_ids, attention_mask, position_ids, weight):
    b, s = input_ids.shape
    n = b * s
    ids = input_ids.reshape(NUM_WORKERS, n // NUM_WORKERS // CHUNK, CHUNK)
    ids = ids.astype(jnp.int32)
    out = _make_gather(n)(ids, weight)
    return (out.reshape(b, s, D_MODEL), attention_mask, position_ids)


# speedup vs baseline: 1.6767x; 1.0312x over previous
import functools

import jax
import jax.numpy as jnp
from jax import lax
from jax.experimental import pallas as pl
from jax.experimental.pallas import tpu as pltpu
from jax.experimental.pallas import tpu_sc as plsc

D_MODEL = 1024
NUM_CORES = 2
NUM_SUBCORES = 16
NUM_WORKERS = NUM_CORES * NUM_SUBCORES
CHUNK = 32
NBUF = 3


def _make_gather(b: int, s: int, io_dtype):
    n_rows = b * s
    per_w = n_rows // NUM_WORKERS
    n_chunks = per_w // CHUNK
    w_per_row = s // per_w
    mesh = plsc.VectorSubcoreMesh(core_axis_name="c", subcore_axis_name="s")

    @functools.partial(
        pl.kernel,
        mesh=mesh,
        out_type=(
            jax.ShapeDtypeStruct((n_rows, D_MODEL), jnp.float32),
            jax.ShapeDtypeStruct((b, s), io_dtype),
            jax.ShapeDtypeStruct((b, s), io_dtype),
        ),
        scratch_types=[
            pltpu.VMEM((per_w,), jnp.int32),
            pltpu.VMEM((NBUF, CHUNK, D_MODEL), jnp.float32),
            pltpu.SemaphoreType.DMA((NBUF,)),
            pltpu.SemaphoreType.DMA((NBUF,)),
            pltpu.SemaphoreType.DMA((2,)),
        ],
    )
    def gather_kernel(ids_hbm, table_hbm, mask_hbm, pos_hbm,
                      out_hbm, mask_out, pos_out,
                      idx_v, rows_v, gsem, osem, psem):
        wid = lax.axis_index("s") * NUM_CORES + lax.axis_index("c")
        base = wid * per_w
        row = wid // w_per_row
        col = (wid % w_per_row) * per_w
        pltpu.sync_copy(ids_hbm.at[row, pl.ds(col, per_w)], idx_v)
        mask_cp = pltpu.make_async_copy(mask_hbm, mask_out, psem.at[0])
        pos_cp = pltpu.make_async_copy(pos_hbm, pos_out, psem.at[1])
        @pl.when(wid == 0)
        def _():
            mask_cp.start()
        @pl.when(wid == 1)
        def _():
            pos_cp.start()
        g_copies = [None] * n_chunks
        o_copies = [None] * n_chunks
        lead = NBUF - 1
        for i in range(n_chunks + lead):
            if i < n_chunks:
                bi = i % NBUF
                if i >= NBUF:
                    o_copies[i - NBUF].wait()
                g_copies[i] = pltpu.async_copy(
                    table_hbm.at[idx_v.at[pl.ds(i * CHUNK, CHUNK)]],
                    rows_v.at[bi], gsem.at[bi])
            if i >= lead:
                j = i - lead
                g_copies[j].wait()
                o_copies[j] = pltpu.async_copy(
                    rows_v.at[j % NBUF],
                    out_hbm.at[pl.ds(base + j * CHUNK, CHUNK)],
                    osem.at[j % NBUF])
        for j in range(n_chunks - NBUF, n_chunks):
            o_copies[j].wait()
        @pl.when(wid == 0)
        def _():
            mask_cp.wait()
        @pl.when(wid == 1)
        def _():
            pos_cp.wait()

    return gather_kernel


def kernel(input_ids, attention_mask, position_ids, weight):
    b, s = input_ids.shape
    ids = input_ids.astype(jnp.int32)
    out, mask, pos = _make_gather(b, s, attention_mask.dtype)(
        ids, weight, attention_mask, position_ids)
    return (out.reshape(b, s, D_MODEL), mask, pos)
